# Initial kernel scaffold; baseline (speedup 1.0000x reference)
import jax
import jax.numpy as jnp
from jax.experimental import pallas as pl


def kernel(init_idxs, domain_idxs, attr_idx, domain_mask, in_W, out_W, out_B, attr_W):
    raise NotImplementedError("write your pallas kernel here")


# speedup vs baseline: 5.1797x; 5.1797x over previous
import functools

import jax
import jax.numpy as jnp
from jax import lax
from jax.experimental import pallas as pl
from jax.experimental.pallas import tpu as pltpu
from jax.experimental.pallas import tpu_sc as plsc

B = 4096
N_CTX = 25
MAX_DOM = 50
D = 64

NC = 2
NS = 16
LANES = 16
NW = NC * NS
B_PER_W = B // NW
GROUPS = B_PER_W // LANES

CHUNK = 80
INIT_CH = (LANES * N_CTX) // CHUNK
DOM_CH = (LANES * MAX_DOM) // CHUNK


def _sc_body(init_hbm, dom_hbm, attr_hbm, mask_hbm, inw_hbm, outw_hbm,
             outb_hbm, attrw_hbm, out_hbm,
             init_v, dom_v, attr_v, a_v, t_v, bias_v, aw_v, mask_v,
             ctx_v, log_v, sem):
    wid = lax.axis_index("s") * NC + lax.axis_index("c")
    iota = lax.iota(jnp.int32, LANES)
    row25 = iota * N_CTX
    row50 = iota * MAX_DOM
    inv_nctx = jnp.float32(1.0 / N_CTX)

    def group_body(g, carry):
        b0 = wid * B_PER_W + g * LANES

        pltpu.sync_copy(init_hbm.at[pl.ds(b0 * N_CTX, LANES * N_CTX)],
                        init_v)
        pltpu.sync_copy(dom_hbm.at[pl.ds(b0 * MAX_DOM, LANES * MAX_DOM)],
                        dom_v)
        pltpu.sync_copy(attr_hbm.at[pl.ds(b0, LANES)], attr_v)
        pltpu.sync_copy(mask_hbm.at[pl.ds(b0 * MAX_DOM, LANES * MAX_DOM)],
                        mask_v)

        copies = []
        for i in range(INIT_CH):
            copies.append(pltpu.async_copy(
                inw_hbm.at[init_v.at[pl.ds(i * CHUNK, CHUNK)]],
                a_v.at[pl.ds(i * CHUNK, CHUNK)], sem))
        for i in range(DOM_CH):
            copies.append(pltpu.async_copy(
                outw_hbm.at[dom_v.at[pl.ds(i * CHUNK, CHUNK)]],
                t_v.at[pl.ds(i * CHUNK, CHUNK)], sem))
        for i in range(DOM_CH):
            copies.append(pltpu.async_copy(
                outb_hbm.at[dom_v.at[pl.ds(i * CHUNK, CHUNK)]],
                bias_v.at[pl.ds(i * CHUNK, CHUNK)], sem))
        copies.append(pltpu.async_copy(attrw_hbm.at[attr_v], aw_v, sem))
        for c in copies:
            c.wait()

        def ctx_body(d, _):
            cold = jnp.full((LANES,), d, dtype=jnp.int32)
            acc = plsc.load_gather(a_v, [row25, cold])
            for j in range(1, N_CTX):
                acc = acc + plsc.load_gather(a_v, [row25 + j, cold])
            aw = plsc.load_gather(aw_v, [iota, cold])
            ctx_v[pl.ds(d * LANES, LANES)] = acc * inv_nctx * aw
            return 0

        lax.fori_loop(0, D, ctx_body, 0, unroll=False)

        kt = 10
        for kc in range(MAX_DOM // kt):
            rows = [row50 + (kc * kt + t) for t in range(kt)]

            def dot_body(d, accs):
                cold = jnp.full((LANES,), d, dtype=jnp.int32)
                c = ctx_v[pl.ds(d * LANES, LANES)]
                return tuple(
                    accs[t] + plsc.load_gather(t_v, [rows[t], cold]) * c
                    for t in range(kt))

            accs = lax.fori_loop(
                0, D, dot_body,
                tuple(jnp.zeros((LANES,), jnp.float32) for _ in range(kt)),
                unroll=False)
            for t in range(kt):
                idxf = rows[t]
                val = (accs[t] + plsc.load_gather(bias_v, [idxf])
                       + plsc.load_gather(mask_v, [idxf]))
                plsc.store_scatter(log_v, [idxf], val)

        pltpu.sync_copy(log_v, out_hbm.at[pl.ds(b0 * MAX_DOM,
                                                LANES * MAX_DOM)])
        return 0

    lax.fori_loop(0, GROUPS, group_body, 0, unroll=False)


@jax.jit
def _run(init_flat, dom_flat, attr_idx, mask_flat, in_W, out_W, out_b1,
         attr_W):
    mesh = plsc.VectorSubcoreMesh(core_axis_name="c", subcore_axis_name="s")
    grid_kernel = pl.kernel(
        _sc_body,
        out_type=jax.ShapeDtypeStruct((B * MAX_DOM,), jnp.float32),
        mesh=mesh,
        compiler_params=pltpu.CompilerParams(
            needs_layout_passes=False, use_tc_tiling_on_sc=False),
        scratch_types=[
            pltpu.VMEM((LANES * N_CTX,), jnp.int32),
            pltpu.VMEM((LANES * MAX_DOM,), jnp.int32),
            pltpu.VMEM((LANES,), jnp.int32),
            pltpu.VMEM((LANES * N_CTX, D), jnp.float32),
            pltpu.VMEM((LANES * MAX_DOM, D), jnp.float32),
            pltpu.VMEM((LANES * MAX_DOM,), jnp.float32),
            pltpu.VMEM((LANES, D), jnp.float32),
            pltpu.VMEM((LANES * MAX_DOM,), jnp.float32),
            pltpu.VMEM((D * LANES,), jnp.float32),
            pltpu.VMEM((LANES * MAX_DOM,), jnp.float32),
            pltpu.SemaphoreType.DMA,
        ],
    )
    return grid_kernel(init_flat, dom_flat, attr_idx, mask_flat, in_W,
                       out_W, out_b1, attr_W)


def kernel(init_idxs, domain_idxs, attr_idx, domain_mask, in_W, out_W,
           out_B, attr_W):
    init_flat = init_idxs.astype(jnp.int32).reshape(B * N_CTX)
    dom_flat = domain_idxs.astype(jnp.int32).reshape(B * MAX_DOM)
    attr32 = attr_idx.astype(jnp.int32)
    mask_flat = domain_mask.reshape(B * MAX_DOM)
    out_b1 = out_B.reshape(-1)
    out = _run(init_flat, dom_flat, attr32, mask_flat, in_W, out_W, out_b1,
               attr_W)
    return out.reshape(B, MAX_DOM)


# speedup vs baseline: 5.4480x; 1.0518x over previous
import jax
import jax.numpy as jnp
from jax import lax
from jax.experimental import pallas as pl
from jax.experimental.pallas import tpu as pltpu
from jax.experimental.pallas import tpu_sc as plsc

B = 4096
N_CTX = 25
MAX_DOM = 50
D = 64

NC = 2
NS = 16
LANES = 16
NW = NC * NS
B_PER_W = B // NW
GROUPS = B_PER_W // LANES
KH = 2
K_TASK = MAX_DOM // KH
ROWS_T = LANES * K_TASK

CHUNK = 80
N_CH = ROWS_T // CHUNK


def _sc_body(init_hbm, dom_hbm, attr_hbm, mask_hbm, inw_hbm, outw_hbm,
             outb_hbm, attrw_hbm, out_hbm,
             iidx_v, didx_v, attr_v, mask_v, aw_v, ctx_v, log_v,
             u_v, bias_v, sems):
    wid = lax.axis_index("s") * NC + lax.axis_index("c")
    iota = lax.iota(jnp.int32, LANES)
    row25 = iota * N_CTX
    row50 = iota * MAX_DOM
    inv_nctx = jnp.float32(1.0 / N_CTX)

    pltpu.sync_copy(init_hbm.at[pl.ds(wid * B_PER_W * N_CTX,
                                      B_PER_W * N_CTX)], iidx_v)
    pltpu.sync_copy(dom_hbm.at[pl.ds(wid * B_PER_W * MAX_DOM,
                                     B_PER_W * MAX_DOM)], didx_v)
    pltpu.sync_copy(attr_hbm.at[pl.ds(wid * B_PER_W, B_PER_W)], attr_v)
    pltpu.sync_copy(mask_hbm.at[pl.ds(wid * B_PER_W * MAX_DOM,
                                      B_PER_W * MAX_DOM)], mask_v)
    pltpu.async_copy(attrw_hbm.at[attr_v], aw_v, sems.at[0]).wait()

    def fire_ctx(g):
        cs = []
        for i in range(N_CH):
            cs.append(pltpu.async_copy(
                inw_hbm.at[iidx_v.at[pl.ds(g * ROWS_T + i * CHUNK, CHUNK)]],
                u_v.at[g % 2].at[pl.ds(i * CHUNK, CHUNK)],
                sems.at[g % 2]))
        return cs

    def fire_dom(t):
        cs = []
        for i in range(N_CH):
            idx = didx_v.at[pl.ds(t * ROWS_T + i * CHUNK, CHUNK)]
            cs.append(pltpu.async_copy(
                outw_hbm.at[idx],
                u_v.at[t % 2].at[pl.ds(i * CHUNK, CHUNK)],
                sems.at[t % 2]))
            cs.append(pltpu.async_copy(
                outb_hbm.at[idx],
                bias_v.at[t % 2].at[pl.ds(i * CHUNK, CHUNK)],
                sems.at[t % 2]))
        return cs

    pending = fire_ctx(0)
    for g in range(GROUPS):
        nxt = fire_ctx(g + 1) if g + 1 < GROUPS else fire_dom(0)
        for c in pending:
            c.wait()
        pending = nxt
        ub = u_v.at[g % 2]
        awrow = g * LANES + iota

        def ctx_body(d, _, ub=ub, awrow=awrow, g=g):
            cold = jnp.full((LANES,), d, dtype=jnp.int32)
            acc = plsc.load_gather(ub, [row25, cold])
            for j in range(1, N_CTX):
                acc = acc + plsc.load_gather(ub, [row25 + j, cold])
            aw = plsc.load_gather(aw_v, [awrow, cold])
            ctx_v[pl.ds(d * B_PER_W + g * LANES, LANES)] = \
                acc * inv_nctx * aw
            return 0

        lax.fori_loop(0, D, ctx_body, 0, unroll=False)

    kt = 5
    for t in range(GROUPS * KH):
        g, h = t // KH, t % KH
        nxt = fire_dom(t + 1) if t + 1 < GROUPS * KH else []
        for c in pending:
            c.wait()
        pending = nxt
        ub = u_v.at[t % 2]
        bb = bias_v.at[t % 2]
        for kc in range(K_TASK // kt):
            rows = [row25 + (kc * kt + s) for s in range(kt)]

            def dot_body(d, accs, ub=ub, rows=rows, g=g):
                cold = jnp.full((LANES,), d, dtype=jnp.int32)
                c = ctx_v[pl.ds(d * B_PER_W + g * LANES, LANES)]
                return tuple(
                    accs[s] + plsc.load_gather(ub, [rows[s], cold]) * c
                    for s in range(kt))

            accs = lax.fori_loop(
                0, D, dot_body,
                tuple(jnp.zeros((LANES,), jnp.float32) for _ in range(kt)),
                unroll=False)
            for s in range(kt):
                kk = kc * kt + s
                gidx = g * (LANES * MAX_DOM) + row50 + (h * K_TASK + kk)
                val = (accs[s] + plsc.load_gather(bb, [rows[s]])
                       + plsc.load_gather(mask_v, [gidx]))
                plsc.store_scatter(log_v, [gidx], val)

    pltpu.sync_copy(log_v, out_hbm.at[pl.ds(wid * B_PER_W * MAX_DOM,
                                            B_PER_W * MAX_DOM)])


@jax.jit
def _run(init_flat, dom_flat, attr_idx, mask_flat, in_W, out_W, out_b1,
         attr_W):
    mesh = plsc.VectorSubcoreMesh(core_axis_name="c", subcore_axis_name="s")
    grid_kernel = pl.kernel(
        _sc_body,
        out_type=jax.ShapeDtypeStruct((B * MAX_DOM,), jnp.float32),
        mesh=mesh,
        compiler_params=pltpu.CompilerParams(
            needs_layout_passes=False, use_tc_tiling_on_sc=False),
        scratch_types=[
            pltpu.VMEM((B_PER_W * N_CTX,), jnp.int32),
            pltpu.VMEM((B_PER_W * MAX_DOM,), jnp.int32),
            pltpu.VMEM((B_PER_W,), jnp.int32),
            pltpu.VMEM((B_PER_W * MAX_DOM,), jnp.float32),
            pltpu.VMEM((B_PER_W, D), jnp.float32),
            pltpu.VMEM((D * B_PER_W,), jnp.float32),
            pltpu.VMEM((B_PER_W * MAX_DOM,), jnp.float32),
            pltpu.VMEM((2, ROWS_T, D), jnp.float32),
            pltpu.VMEM((2, ROWS_T), jnp.float32),
            pltpu.SemaphoreType.DMA((2,)),
        ],
    )
    return grid_kernel(init_flat, dom_flat, attr_idx, mask_flat, in_W,
                       out_W, out_b1, attr_W)


def kernel(init_idxs, domain_idxs, attr_idx, domain_mask, in_W, out_W,
           out_B, attr_W):
    init_flat = init_idxs.astype(jnp.int32).reshape(B * N_CTX)
    dom_flat = (domain_idxs.astype(jnp.int32)
                .reshape(NW, GROUPS, LANES, KH, K_TASK)
                .transpose(0, 1, 3, 2, 4)
                .reshape(B * MAX_DOM))
    attr32 = attr_idx.astype(jnp.int32)
    mask_flat = domain_mask.reshape(B * MAX_DOM)
    out_b1 = out_B.reshape(-1)
    out = _run(init_flat, dom_flat, attr32, mask_flat, in_W, out_W, out_b1,
               attr_W)
    return out.reshape(B, MAX_DOM)


# speedup vs baseline: 13.5122x; 2.4802x over previous
import jax
import jax.numpy as jnp
from jax import lax
from jax.experimental import pallas as pl
from jax.experimental.pallas import tpu as pltpu
from jax.experimental.pallas import tpu_sc as plsc

B = 4096
N_CTX = 25
MAX_DOM = 50
D = 64

NC = 2
NS = 16
LANES = 16
NW = NC * NS
B_PER_W = B // NW
GROUPS = B_PER_W // LANES
KH = 2
K_TASK = MAX_DOM // KH
ROWS_T = LANES * K_TASK

CHUNK = 80
N_CH = ROWS_T // CHUNK


def _sc_body(init_hbm, dom_hbm, attr_hbm, mask_hbm, inw_hbm, outw_hbm,
             outb_hbm, attrw_hbm, out_hbm,
             iidx_v, didx_v, attr_v, mask_v, aw_v, ctx_v, log_v,
             u_v, bias_v, sems):
    wid = lax.axis_index("s") * NC + lax.axis_index("c")
    iota = lax.iota(jnp.int32, LANES)
    row25 = iota * N_CTX
    row50 = iota * MAX_DOM
    inv_nctx = jnp.float32(1.0 / N_CTX)

    pltpu.sync_copy(init_hbm.at[pl.ds(wid * B_PER_W * N_CTX,
                                      B_PER_W * N_CTX)], iidx_v)
    pltpu.sync_copy(dom_hbm.at[pl.ds(wid * B_PER_W * MAX_DOM,
                                     B_PER_W * MAX_DOM)], didx_v)
    pltpu.sync_copy(attr_hbm.at[pl.ds(wid * B_PER_W, B_PER_W)], attr_v)
    pltpu.sync_copy(mask_hbm.at[pl.ds(wid * B_PER_W * MAX_DOM,
                                      B_PER_W * MAX_DOM)], mask_v)
    pltpu.async_copy(attrw_hbm.at[attr_v], aw_v, sems.at[0]).wait()

    NT = GROUPS + GROUPS * KH

    def fire(t):
        p = t & 1

        def f_ctx():
            for i in range(N_CH):
                pltpu.async_copy(
                    inw_hbm.at[iidx_v.at[pl.ds(t * ROWS_T + i * CHUNK,
                                               CHUNK)]],
                    u_v.at[p].at[pl.ds(i * CHUNK, CHUNK)],
                    sems.at[p])

        def f_dom():
            tt = t - GROUPS
            for i in range(N_CH):
                idx = didx_v.at[pl.ds(tt * ROWS_T + i * CHUNK, CHUNK)]
                pltpu.async_copy(
                    outw_hbm.at[idx],
                    u_v.at[p].at[pl.ds(i * CHUNK, CHUNK)],
                    sems.at[p])
                pltpu.async_copy(
                    outb_hbm.at[idx],
                    bias_v.at[p].at[pl.ds(i * CHUNK, CHUNK)],
                    sems.at[p])

        lax.cond(t < GROUPS, f_ctx, f_dom)

    def drain(t):
        p = t & 1

        def d_ctx():
            pltpu.make_async_copy(inw_hbm.at[pl.ds(0, ROWS_T)],
                                  u_v.at[p], sems.at[p]).wait()

        def d_dom():
            pltpu.make_async_copy(outw_hbm.at[pl.ds(0, ROWS_T)],
                                  u_v.at[p], sems.at[p]).wait()
            pltpu.make_async_copy(outb_hbm.at[pl.ds(0, ROWS_T)],
                                  bias_v.at[p], sems.at[p]).wait()

        lax.cond(t < GROUPS, d_ctx, d_dom)

    def ctx_task(g):
        ub = u_v.at[g & 1]
        awrow = g * LANES + iota

        def ctx_body(d, _):
            cold = ((d + iota) & (LANES - 1)) | (d & ~(LANES - 1))
            acc = plsc.load_gather(ub, [row25, cold])
            for j in range(1, N_CTX):
                acc = acc + plsc.load_gather(ub, [row25 + j, cold])
            aw = plsc.load_gather(aw_v, [awrow, cold])
            ctx_v[pl.ds(d * B_PER_W + g * LANES, LANES)] = \
                acc * inv_nctx * aw
            return 0

        lax.fori_loop(0, D, ctx_body, 0, unroll=False)

    kt = 5

    def dom_task(tt):
        g = tt >> 1
        h = tt & 1
        ub = u_v.at[tt & 1]
        bb = bias_v.at[tt & 1]
        for kc in range(K_TASK // kt):
            rows = [row25 + (kc * kt + s) for s in range(kt)]

            def dot_body(d, accs, rows=rows):
                cold = ((d + iota) & (LANES - 1)) | (d & ~(LANES - 1))
                c = ctx_v[pl.ds(d * B_PER_W + g * LANES, LANES)]
                return tuple(
                    accs[s] + plsc.load_gather(ub, [rows[s], cold]) * c
                    for s in range(kt))

            accs = lax.fori_loop(
                0, D, dot_body,
                tuple(jnp.zeros((LANES,), jnp.float32) for _ in range(kt)),
                unroll=False)
            for s in range(kt):
                kk = kc * kt + s
                gidx = (g * (LANES * MAX_DOM) + row50
                        + (h * K_TASK + kk))
                val = (accs[s] + plsc.load_gather(bb, [rows[s]])
                       + plsc.load_gather(mask_v, [gidx]))
                plsc.store_scatter(log_v, [gidx], val)

    fire(0)

    def task_body(t, _):
        lax.cond(t + 1 < NT, lambda: fire(t + 1), lambda: None)
        drain(t)
        lax.cond(t < GROUPS,
                 lambda: ctx_task(t),
                 lambda: dom_task(t - GROUPS))
        return 0

    lax.fori_loop(0, NT, task_body, 0, unroll=False)

    pltpu.sync_copy(log_v, out_hbm.at[pl.ds(wid * B_PER_W * MAX_DOM,
                                            B_PER_W * MAX_DOM)])


@jax.jit
def _run(init_flat, dom_flat, attr_idx, mask_flat, in_W, out_W, out_b1,
         attr_W):
    mesh = plsc.VectorSubcoreMesh(core_axis_name="c", subcore_axis_name="s")
    grid_kernel = pl.kernel(
        _sc_body,
        out_type=jax.ShapeDtypeStruct((B * MAX_DOM,), jnp.float32),
        mesh=mesh,
        compiler_params=pltpu.CompilerParams(
            needs_layout_passes=False, use_tc_tiling_on_sc=False),
        scratch_types=[
            pltpu.VMEM((B_PER_W * N_CTX,), jnp.int32),
            pltpu.VMEM((B_PER_W * MAX_DOM,), jnp.int32),
            pltpu.VMEM((B_PER_W,), jnp.int32),
            pltpu.VMEM((B_PER_W * MAX_DOM,), jnp.float32),
            pltpu.VMEM((B_PER_W, D), jnp.float32),
            pltpu.VMEM((D * B_PER_W,), jnp.float32),
            pltpu.VMEM((B_PER_W * MAX_DOM,), jnp.float32),
            pltpu.VMEM((2, ROWS_T, D), jnp.float32),
            pltpu.VMEM((2, ROWS_T), jnp.float32),
            pltpu.SemaphoreType.DMA((2,)),
        ],
    )
    return grid_kernel(init_flat, dom_flat, attr_idx, mask_flat, in_W,
                       out_W, out_b1, attr_W)


def kernel(init_idxs, domain_idxs, attr_idx, domain_mask, in_W, out_W,
           out_B, attr_W):
    init_flat = init_idxs.astype(jnp.int32).reshape(B * N_CTX)
    dom_flat = (domain_idxs.astype(jnp.int32)
                .reshape(NW, GROUPS, LANES, KH, K_TASK)
                .transpose(0, 1, 3, 2, 4)
                .reshape(B * MAX_DOM))
    attr32 = attr_idx.astype(jnp.int32)
    mask_flat = domain_mask.reshape(B * MAX_DOM)
    out_b1 = out_B.reshape(-1)
    out = _run(init_flat, dom_flat, attr32, mask_flat, in_W, out_W, out_b1,
               attr_W)
    return out.reshape(B, MAX_DOM)
